# baseline (speedup 1.0000x reference)
import functools

import jax
import jax.numpy as jnp
from jax import lax
from jax.experimental import pallas as pl
from jax.experimental.pallas import tpu as pltpu
from jax.experimental.pallas import tpu_sc as plsc

N = 200000
CA = 128
CM = 128
B = 8
NW = 32

CHUNK = 160
NCHUNKS = N // CHUNK

CHUNK2 = 320
NCHUNKS2 = N // CHUNK2

_SC_PARAMS = pltpu.CompilerParams(needs_layout_passes=False)


def _sc_segsum(img, flow, bids, wsp1d, bsp16):
    mesh = plsc.VectorSubcoreMesh(core_axis_name="c", subcore_axis_name="s")

    @functools.partial(
        pl.kernel,
        mesh=mesh,
        out_type=[jax.ShapeDtypeStruct((NW, 16, CA), jnp.float32),
                  jax.ShapeDtypeStruct((N,), jnp.float32)],
        scratch_types=[
            pltpu.VMEM((2 * CHUNK, CA), jnp.float32),
            pltpu.VMEM((2 * CHUNK, CM), jnp.float32),
            pltpu.VMEM((2 * CHUNK,), jnp.float32),
            pltpu.VMEM((2 * CHUNK,), jnp.int32),
            pltpu.VMEM((16, CA), jnp.float32),
            pltpu.VMEM((CM,), jnp.float32),
            pltpu.VMEM((16,), jnp.float32),
            pltpu.SemaphoreType.DMA,
            pltpu.SemaphoreType.DMA,
        ],
        compiler_params=_SC_PARAMS,
    )
    def seg_kernel(img_hbm, flow_hbm, bids_hbm, wsp_hbm, bsp_hbm,
                   out_hbm, gate_hbm, img_v, flow_v, gate_v, bids_v,
                   acc_v, wsp_v, bsp_v, sem_in, sem_gout):
        wid = lax.axis_index("s") * 2 + lax.axis_index("c")
        zero16 = jnp.zeros((16,), jnp.float32)
        for i in range(16):
            for k in range(CA // 16):
                acc_v[i, pl.ds(k * 16, 16)] = zero16
        nch = (NCHUNKS // NW) + (wid < (NCHUNKS % NW)).astype(jnp.int32)
        iota16 = lax.iota(jnp.int32, 16)
        one16 = jnp.ones((16,), jnp.float32)
        zro16 = jnp.zeros((16,), jnp.float32)
        zaccs = tuple(zro16 for _ in range(CA // 16))
        pltpu.sync_copy(wsp_hbm, wsp_v)
        pltpu.sync_copy(bsp_hbm, bsp_v)
        wspk = [wsp_v[pl.ds(k * 16, 16)] for k in range(CM // 16)]
        bspv = bsp_v[pl.ds(0, 16)]

        def in_copies(t, slot):
            base = (t * NW + wid) * CHUNK
            return (
                pltpu.make_async_copy(img_hbm.at[pl.ds(base, CHUNK), :],
                                      img_v.at[pl.ds(slot * CHUNK, CHUNK), :],
                                      sem_in),
                pltpu.make_async_copy(flow_hbm.at[pl.ds(base, CHUNK), :],
                                      flow_v.at[pl.ds(slot * CHUNK, CHUNK), :],
                                      sem_in),
                pltpu.make_async_copy(bids_hbm.at[pl.ds(base, CHUNK)],
                                      bids_v.at[pl.ds(slot * CHUNK, CHUNK)],
                                      sem_in),
            )

        def gout_copy(t, slot):
            base = (t * NW + wid) * CHUNK
            return pltpu.make_async_copy(
                gate_v.at[pl.ds(slot * CHUNK, CHUNK)],
                gate_hbm.at[pl.ds(base, CHUNK)], sem_gout)

        pltpu.sync_copy(bids_hbm.at[pl.ds(wid * CHUNK, 16)],
                        bids_v.at[pl.ds(0, 16)])
        cur0 = bids_v[pl.ds(0, 16)][0]

        for c in in_copies(0, 0):
            c.start()

        def flush(accs, seg):
            for k in range(CA // 16):
                plsc.addupdate(acc_v.at[seg, pl.ds(k * 16, 16)], accs[k])

        def chunk_body(t, st):
            accs, counts, cur = st
            slot = lax.rem(t, 2)
            soff = slot * CHUNK
            for c in in_copies(t, slot):
                c.wait()

            @pl.when(t + 1 < nch)
            def _():
                for c in in_copies(t + 1, 1 - slot):
                    c.start()

            @pl.when(t >= 2)
            def _():
                gout_copy(t - 2, slot).wait()

            def grp_body(q, st):
                accs, counts, cur = st
                bvec = bids_v[pl.ds(soff + q * 16, 16)]
                row0 = soff + q * 16
                dvec = zro16
                for j in range(16):
                    dv = flow_v[row0 + j, pl.ds(0, 16)] * wspk[0]
                    for k in range(1, CM // 16):
                        dv = dv + flow_v[row0 + j, pl.ds(k * 16, 16)] * wspk[k]
                    dj = lax.reduce_sum(dv, axes=(0,))
                    oh = jnp.where(iota16 == j, one16, zro16)
                    dvec = dvec + lax.broadcast_in_dim(dj, (16,), ()) * oh
                gvec = 1.0 / (1.0 + jnp.exp(-(dvec + bspv)))
                gate_v[pl.ds(soff + q * 16, 16)] = (
                    gvec + bvec.astype(jnp.float32))
                b0 = bvec[0]
                b15 = bvec[15]
                mixed = b0 != b15
                fcond = (b0 != cur) | mixed
                ffac = jnp.where(fcond, 1.0, 0.0)
                mfac = jnp.where(mixed, 1.0, 0.0)
                ffac_b = lax.broadcast_in_dim(ffac, (16,), ())
                keep_b = lax.broadcast_in_dim(1.0 - ffac, (16,), ())

                a = []
                for k in range(CA // 16):
                    plsc.addupdate(acc_v.at[cur, pl.ds(k * 16, 16)],
                                   accs[k] * ffac_b)
                    a.append(accs[k] * keep_b)
                cur = jnp.where(fcond, b0, cur)

                gmask = gvec * lax.broadcast_in_dim(1.0 - mfac, (16,), ())
                for j in range(16):
                    gb = lax.broadcast_in_dim(gmask[j], (16,), ())
                    bb = lax.broadcast_in_dim(bvec[j], (16,), ())
                    for k in range(CA // 16):
                        a[k] = a[k] + img_v[row0 + j, pl.ds(k * 16, 16)] * gb
                    counts = counts + jnp.where(iota16 == bb, one16, zro16)

                @pl.when(mixed)
                def _():
                    for j in range(16):
                        gb = lax.broadcast_in_dim(gvec[j], (16,), ())
                        for k in range(CA // 16):
                            v = img_v[row0 + j, pl.ds(k * 16, 16)] * gb
                            plsc.addupdate(acc_v.at[bvec[j], pl.ds(k * 16, 16)],
                                           v)

                cur = jnp.where(mixed, b15, cur)
                return (tuple(a), counts, cur)

            st = lax.fori_loop(0, CHUNK // 16, grp_body, (accs, counts, cur))
            gout_copy(t, slot).start()
            return st

        accs, counts, cur = lax.fori_loop(
            0, nch, chunk_body,
            (zaccs, jnp.zeros((16,), jnp.float32), cur0))
        flush(accs, cur)
        acc_v[B, pl.ds(0, 16)] = counts
        pltpu.sync_copy(acc_v, out_hbm.at[wid])
        gout_copy(nch - 2, lax.rem(nch - 2, 2)).wait()
        gout_copy(nch - 1, lax.rem(nch - 1, 2)).wait()

    return seg_kernel(img, flow, bids, wsp1d, bsp16)


def _fv_body(p_ref, wch_ref, bch_ref, fv_ref):
    t = p_ref[0]
    for i in range(1, NW):
        t = t + p_ref[i]
    sums = t[0:B, :]
    craw = t[B:B + 1, 0:16]
    sel = (lax.broadcasted_iota(jnp.int32, (B, 16), 0)
           == lax.broadcasted_iota(jnp.int32, (B, 16), 1)).astype(jnp.float32)
    counts = lax.dot_general(sel, craw, (((1,), (1,)), ((), ())),
                             preferred_element_type=jnp.float32)
    mean = sums / jnp.maximum(counts, 1.0)
    z = lax.dot_general(mean, wch_ref[...], (((1,), (1,)), ((), ())),
                        preferred_element_type=jnp.float32) + bch_ref[...]
    z = z - jnp.max(z, axis=1, keepdims=True)
    e = jnp.exp(z)
    fv_ref[...] = e / jnp.sum(e, axis=1, keepdims=True) * CA


def _fv_call(partials, wch, bch):
    return pl.pallas_call(
        _fv_body,
        in_specs=[
            pl.BlockSpec((NW, 16, CA), lambda: (0, 0, 0)),
            pl.BlockSpec((CA, CA), lambda: (0, 0)),
            pl.BlockSpec((1, CA), lambda: (0, 0)),
        ],
        out_specs=pl.BlockSpec((B, CA), lambda: (0, 0)),
        out_shape=jax.ShapeDtypeStruct((B, CA), jnp.float32),
    )(partials, wch, bch)


def _sc_map(img, pk, fv):
    mesh = plsc.VectorSubcoreMesh(core_axis_name="c", subcore_axis_name="s")

    @functools.partial(
        pl.kernel,
        mesh=mesh,
        out_type=jax.ShapeDtypeStruct((N, CA), jnp.float32),
        scratch_types=[
            pltpu.VMEM((2 * CHUNK2, CA), jnp.float32),
            pltpu.VMEM((CHUNK2, CA), jnp.float32),
            pltpu.VMEM((2 * CHUNK2,), jnp.float32),
            pltpu.VMEM((B, CA), jnp.float32),
            pltpu.SemaphoreType.DMA,
            pltpu.SemaphoreType.DMA,
        ],
        compiler_params=_SC_PARAMS,
    )
    def map_kernel(img_hbm, pk_hbm, fv_hbm, out_hbm,
                   img_v, out_v, pk_v, fv_v, sem_in, sem_out):
        wid = lax.axis_index("s") * 2 + lax.axis_index("c")
        nch = (NCHUNKS2 // NW) + (wid < (NCHUNKS2 % NW)).astype(jnp.int32)
        pltpu.sync_copy(fv_hbm, fv_v)

        def in_copies(t, slot):
            base = (t * NW + wid) * CHUNK2
            return (
                pltpu.make_async_copy(img_hbm.at[pl.ds(base, CHUNK2), :],
                                      img_v.at[pl.ds(slot * CHUNK2, CHUNK2), :],
                                      sem_in),
                pltpu.make_async_copy(pk_hbm.at[pl.ds(base, CHUNK2)],
                                      pk_v.at[pl.ds(slot * CHUNK2, CHUNK2)],
                                      sem_in),
            )

        def out_copy(t):
            base = (t * NW + wid) * CHUNK2
            return pltpu.make_async_copy(
                out_v, out_hbm.at[pl.ds(base, CHUNK2), :], sem_out)

        for c in in_copies(0, 0):
            c.start()

        def chunk_body(t, _):
            slot = lax.rem(t, 2)
            soff = slot * CHUNK2
            irow = slot * CHUNK2
            for c in in_copies(t, slot):
                c.wait()

            @pl.when(t + 1 < nch)
            def _():
                for c in in_copies(t + 1, 1 - slot):
                    c.start()

            @pl.when(t >= 1)
            def _():
                out_copy(t - 1).wait()

            def grp_body(q, _):
                pkvec = pk_v[pl.ds(soff + q * 16, 16)]
                bvec = pkvec.astype(jnp.int32)
                gvec = pkvec - bvec.astype(jnp.float32)
                row0 = irow + q * 16
                orow = q * 16
                b0 = bvec[0]
                b15 = bvec[15]
                mixed = b0 != b15
                fvk = [fv_v[b0, pl.ds(k * 16, 16)] for k in range(CA // 16)]
                for j in range(16):
                    gb = lax.broadcast_in_dim(gvec[j], (16,), ())
                    for k in range(CA // 16):
                        v = img_v[row0 + j, pl.ds(k * 16, 16)]
                        out_v[orow + j, pl.ds(k * 16, 16)] = (
                            v * (gb * fvk[k] + 1.0))

                @pl.when(mixed)
                def _():
                    for j in range(16):
                        gb = lax.broadcast_in_dim(gvec[j], (16,), ())
                        for k in range(CA // 16):
                            v = img_v[row0 + j, pl.ds(k * 16, 16)]
                            f = fv_v[bvec[j], pl.ds(k * 16, 16)]
                            out_v[orow + j, pl.ds(k * 16, 16)] = (
                                v * (gb * f + 1.0))
                return 0

            lax.fori_loop(0, CHUNK2 // 16, grp_body, 0)
            out_copy(t).start()
            return 0

        lax.fori_loop(0, nch, chunk_body, 0)
        out_copy(nch - 1).wait()

    return map_kernel(img, pk, fv)


def kernel(img_feat_features, flow_feat_features, batch_ids, W_sp, b_sp, W_ch, b_ch):
    img = img_feat_features
    flow = flow_feat_features
    bids = batch_ids.astype(jnp.int32)
    wsp1d = W_sp.reshape(CM)
    bsp16 = jnp.broadcast_to(b_sp, (16,))
    bch = b_ch.reshape(1, CA)

    partials, pk = _sc_segsum(img, flow, bids, wsp1d, bsp16)
    fv = _fv_call(partials, W_ch, bch)
    out = _sc_map(img, pk, fv)
    return out


# speedup vs baseline: 1.0948x; 1.0948x over previous
import functools

import jax
import jax.numpy as jnp
from jax import lax
from jax.experimental import pallas as pl
from jax.experimental.pallas import tpu as pltpu
from jax.experimental.pallas import tpu_sc as plsc

N = 200000
CA = 128
CM = 128
B = 8
NW = 32

CHUNK = 160
NCHUNKS = N // CHUNK

CHUNK2 = 160
NCHUNKS2 = N // CHUNK2

_SC_PARAMS = pltpu.CompilerParams(needs_layout_passes=False)


def _sc_segsum(img, flow, bids, wsp1d, bsp16):
    mesh = plsc.VectorSubcoreMesh(core_axis_name="c", subcore_axis_name="s")

    @functools.partial(
        pl.kernel,
        mesh=mesh,
        out_type=[jax.ShapeDtypeStruct((NW, 16, CA), jnp.float32),
                  jax.ShapeDtypeStruct((N,), jnp.float32)],
        scratch_types=[
            pltpu.VMEM((2 * CHUNK, CA), jnp.float32),
            pltpu.VMEM((2 * CHUNK, CM), jnp.float32),
            pltpu.VMEM((2 * CHUNK,), jnp.float32),
            pltpu.VMEM((2 * CHUNK,), jnp.int32),
            pltpu.VMEM((16, CA), jnp.float32),
            pltpu.VMEM((CM,), jnp.float32),
            pltpu.VMEM((16,), jnp.float32),
            pltpu.SemaphoreType.DMA,
            pltpu.SemaphoreType.DMA,
        ],
        compiler_params=_SC_PARAMS,
    )
    def seg_kernel(img_hbm, flow_hbm, bids_hbm, wsp_hbm, bsp_hbm,
                   out_hbm, gate_hbm, img_v, flow_v, gate_v, bids_v,
                   acc_v, wsp_v, bsp_v, sem_in, sem_gout):
        wid = lax.axis_index("s") * 2 + lax.axis_index("c")
        zero16 = jnp.zeros((16,), jnp.float32)
        for i in range(16):
            for k in range(CA // 16):
                acc_v[i, pl.ds(k * 16, 16)] = zero16
        nch = (NCHUNKS // NW) + (wid < (NCHUNKS % NW)).astype(jnp.int32)
        iota16 = lax.iota(jnp.int32, 16)
        one16 = jnp.ones((16,), jnp.float32)
        zro16 = jnp.zeros((16,), jnp.float32)
        zaccs = tuple(zro16 for _ in range(CA // 16))
        pltpu.sync_copy(wsp_hbm, wsp_v)
        pltpu.sync_copy(bsp_hbm, bsp_v)
        wspk = [wsp_v[pl.ds(k * 16, 16)] for k in range(CM // 16)]
        bspv = bsp_v[pl.ds(0, 16)]

        def in_copies(t, slot):
            base = (t * NW + wid) * CHUNK
            return (
                pltpu.make_async_copy(img_hbm.at[pl.ds(base, CHUNK), :],
                                      img_v.at[pl.ds(slot * CHUNK, CHUNK), :],
                                      sem_in),
                pltpu.make_async_copy(flow_hbm.at[pl.ds(base, CHUNK), :],
                                      flow_v.at[pl.ds(slot * CHUNK, CHUNK), :],
                                      sem_in),
                pltpu.make_async_copy(bids_hbm.at[pl.ds(base, CHUNK)],
                                      bids_v.at[pl.ds(slot * CHUNK, CHUNK)],
                                      sem_in),
            )

        def gout_copy(t, slot):
            base = (t * NW + wid) * CHUNK
            return pltpu.make_async_copy(
                gate_v.at[pl.ds(slot * CHUNK, CHUNK)],
                gate_hbm.at[pl.ds(base, CHUNK)], sem_gout)

        pltpu.sync_copy(bids_hbm.at[pl.ds(wid * CHUNK, 16)],
                        bids_v.at[pl.ds(0, 16)])
        cur0 = bids_v[pl.ds(0, 16)][0]

        for c in in_copies(0, 0):
            c.start()

        def flush(accs, seg):
            for k in range(CA // 16):
                plsc.addupdate(acc_v.at[seg, pl.ds(k * 16, 16)], accs[k])

        def chunk_body(t, st):
            accs, counts, cur = st
            slot = lax.rem(t, 2)
            soff = slot * CHUNK
            for c in in_copies(t, slot):
                c.wait()

            @pl.when(t + 1 < nch)
            def _():
                for c in in_copies(t + 1, 1 - slot):
                    c.start()

            @pl.when(t >= 2)
            def _():
                gout_copy(t - 2, slot).wait()

            def grp_body(q, st):
                accs, counts, cur = st
                bvec = bids_v[pl.ds(soff + q * 16, 16)]
                row0 = soff + q * 16
                dvec = zro16
                for j in range(16):
                    dv = flow_v[row0 + j, pl.ds(0, 16)] * wspk[0]
                    for k in range(1, CM // 16):
                        dv = dv + flow_v[row0 + j, pl.ds(k * 16, 16)] * wspk[k]
                    dj = lax.reduce_sum(dv, axes=(0,))
                    oh = jnp.where(iota16 == j, one16, zro16)
                    dvec = dvec + lax.broadcast_in_dim(dj, (16,), ()) * oh
                gvec = 1.0 / (1.0 + jnp.exp(-(dvec + bspv)))
                gate_v[pl.ds(soff + q * 16, 16)] = (
                    gvec + bvec.astype(jnp.float32))
                b0 = bvec[0]
                b15 = bvec[15]
                mixed = b0 != b15
                fcond = (b0 != cur) | mixed
                ffac = jnp.where(fcond, 1.0, 0.0)
                mfac = jnp.where(mixed, 1.0, 0.0)
                ffac_b = lax.broadcast_in_dim(ffac, (16,), ())
                keep_b = lax.broadcast_in_dim(1.0 - ffac, (16,), ())

                a = []
                for k in range(CA // 16):
                    plsc.addupdate(acc_v.at[cur, pl.ds(k * 16, 16)],
                                   accs[k] * ffac_b)
                    a.append(accs[k] * keep_b)
                cur = jnp.where(fcond, b0, cur)

                gmask = gvec * lax.broadcast_in_dim(1.0 - mfac, (16,), ())
                for j in range(16):
                    gb = lax.broadcast_in_dim(gmask[j], (16,), ())
                    bb = lax.broadcast_in_dim(bvec[j], (16,), ())
                    for k in range(CA // 16):
                        a[k] = a[k] + img_v[row0 + j, pl.ds(k * 16, 16)] * gb
                    counts = counts + jnp.where(iota16 == bb, one16, zro16)

                @pl.when(mixed)
                def _():
                    for j in range(16):
                        gb = lax.broadcast_in_dim(gvec[j], (16,), ())
                        for k in range(CA // 16):
                            v = img_v[row0 + j, pl.ds(k * 16, 16)] * gb
                            plsc.addupdate(acc_v.at[bvec[j], pl.ds(k * 16, 16)],
                                           v)

                cur = jnp.where(mixed, b15, cur)
                return (tuple(a), counts, cur)

            st = lax.fori_loop(0, CHUNK // 16, grp_body, (accs, counts, cur))
            gout_copy(t, slot).start()
            return st

        accs, counts, cur = lax.fori_loop(
            0, nch, chunk_body,
            (zaccs, jnp.zeros((16,), jnp.float32), cur0))
        flush(accs, cur)
        acc_v[B, pl.ds(0, 16)] = counts
        pltpu.sync_copy(acc_v, out_hbm.at[wid])
        gout_copy(nch - 2, lax.rem(nch - 2, 2)).wait()
        gout_copy(nch - 1, lax.rem(nch - 1, 2)).wait()

    return seg_kernel(img, flow, bids, wsp1d, bsp16)


def _fv_body(p_ref, wch_ref, bch_ref, fv_ref):
    t = p_ref[0]
    for i in range(1, NW):
        t = t + p_ref[i]
    sums = t[0:B, :]
    craw = t[B:B + 1, 0:16]
    sel = (lax.broadcasted_iota(jnp.int32, (B, 16), 0)
           == lax.broadcasted_iota(jnp.int32, (B, 16), 1)).astype(jnp.float32)
    counts = lax.dot_general(sel, craw, (((1,), (1,)), ((), ())),
                             preferred_element_type=jnp.float32)
    mean = sums / jnp.maximum(counts, 1.0)
    z = lax.dot_general(mean, wch_ref[...], (((1,), (1,)), ((), ())),
                        preferred_element_type=jnp.float32) + bch_ref[...]
    z = z - jnp.max(z, axis=1, keepdims=True)
    e = jnp.exp(z)
    fv_ref[...] = e / jnp.sum(e, axis=1, keepdims=True) * CA


def _fv_call(partials, wch, bch):
    return pl.pallas_call(
        _fv_body,
        in_specs=[
            pl.BlockSpec((NW, 16, CA), lambda: (0, 0, 0)),
            pl.BlockSpec((CA, CA), lambda: (0, 0)),
            pl.BlockSpec((1, CA), lambda: (0, 0)),
        ],
        out_specs=pl.BlockSpec((B, CA), lambda: (0, 0)),
        out_shape=jax.ShapeDtypeStruct((B, CA), jnp.float32),
    )(partials, wch, bch)


def _sc_map(img, pk, fv):
    mesh = plsc.VectorSubcoreMesh(core_axis_name="c", subcore_axis_name="s")

    @functools.partial(
        pl.kernel,
        mesh=mesh,
        out_type=jax.ShapeDtypeStruct((N, CA), jnp.float32),
        scratch_types=[
            pltpu.VMEM((2 * CHUNK2, CA), jnp.float32),
            pltpu.VMEM((2 * CHUNK2, CA), jnp.float32),
            pltpu.VMEM((2 * CHUNK2,), jnp.float32),
            pltpu.VMEM((B, CA), jnp.float32),
            pltpu.SemaphoreType.DMA,
            pltpu.SemaphoreType.DMA,
        ],
        compiler_params=_SC_PARAMS,
    )
    def map_kernel(img_hbm, pk_hbm, fv_hbm, out_hbm,
                   img_v, out_v, pk_v, fv_v, sem_in, sem_out):
        wid = lax.axis_index("s") * 2 + lax.axis_index("c")
        nch = (NCHUNKS2 // NW) + (wid < (NCHUNKS2 % NW)).astype(jnp.int32)
        pltpu.sync_copy(fv_hbm, fv_v)

        def in_copies(t, slot):
            base = (t * NW + wid) * CHUNK2
            return (
                pltpu.make_async_copy(img_hbm.at[pl.ds(base, CHUNK2), :],
                                      img_v.at[pl.ds(slot * CHUNK2, CHUNK2), :],
                                      sem_in),
                pltpu.make_async_copy(pk_hbm.at[pl.ds(base, CHUNK2)],
                                      pk_v.at[pl.ds(slot * CHUNK2, CHUNK2)],
                                      sem_in),
            )

        def out_copy(t, slot):
            base = (t * NW + wid) * CHUNK2
            return pltpu.make_async_copy(
                out_v.at[pl.ds(slot * CHUNK2, CHUNK2), :],
                out_hbm.at[pl.ds(base, CHUNK2), :], sem_out)

        for c in in_copies(0, 0):
            c.start()

        def chunk_body(t, _):
            slot = lax.rem(t, 2)
            soff = slot * CHUNK2
            irow = slot * CHUNK2
            for c in in_copies(t, slot):
                c.wait()

            @pl.when(t + 1 < nch)
            def _():
                for c in in_copies(t + 1, 1 - slot):
                    c.start()

            @pl.when(t >= 2)
            def _():
                out_copy(t - 2, slot).wait()

            def grp_body(q, _):
                pkvec = pk_v[pl.ds(soff + q * 16, 16)]
                bvec = pkvec.astype(jnp.int32)
                gvec = pkvec - bvec.astype(jnp.float32)
                row0 = irow + q * 16
                orow = irow + q * 16
                b0 = bvec[0]
                b15 = bvec[15]
                mixed = b0 != b15
                fvk = [fv_v[b0, pl.ds(k * 16, 16)] for k in range(CA // 16)]
                for j in range(16):
                    gb = lax.broadcast_in_dim(gvec[j], (16,), ())
                    for k in range(CA // 16):
                        v = img_v[row0 + j, pl.ds(k * 16, 16)]
                        out_v[orow + j, pl.ds(k * 16, 16)] = (
                            v * (gb * fvk[k] + 1.0))

                @pl.when(mixed)
                def _():
                    for j in range(16):
                        gb = lax.broadcast_in_dim(gvec[j], (16,), ())
                        for k in range(CA // 16):
                            v = img_v[row0 + j, pl.ds(k * 16, 16)]
                            f = fv_v[bvec[j], pl.ds(k * 16, 16)]
                            out_v[orow + j, pl.ds(k * 16, 16)] = (
                                v * (gb * f + 1.0))
                return 0

            lax.fori_loop(0, CHUNK2 // 16, grp_body, 0)
            out_copy(t, slot).start()
            return 0

        lax.fori_loop(0, nch, chunk_body, 0)
        out_copy(nch - 2, lax.rem(nch - 2, 2)).wait()
        out_copy(nch - 1, lax.rem(nch - 1, 2)).wait()

    return map_kernel(img, pk, fv)


def kernel(img_feat_features, flow_feat_features, batch_ids, W_sp, b_sp, W_ch, b_ch):
    img = img_feat_features
    flow = flow_feat_features
    bids = batch_ids.astype(jnp.int32)
    wsp1d = W_sp.reshape(CM)
    bsp16 = jnp.broadcast_to(b_sp, (16,))
    bch = b_ch.reshape(1, CA)

    partials, pk = _sc_segsum(img, flow, bids, wsp1d, bsp16)
    fv = _fv_call(partials, W_ch, bch)
    out = _sc_map(img, pk, fv)
    return out
